# Initial kernel scaffold; baseline (speedup 1.0000x reference)
import functools

import jax
import jax.numpy as jnp
from jax import lax
from jax.experimental import pallas as pl
from jax.experimental.pallas import tpu as pltpu
from jax.experimental.pallas import tpu_sc as plsc

_N_NODES = 10000
_N_EDGES = 320000
_D = 128

_NC = 2
_NS = 16
_NW = _NC * _NS
_EPW = _N_EDGES // _NW
_L = 16


def _pq_body(z_ref, w2_ref, b2_ref, out_ref):
    z = z_ref[...]
    p = jnp.sum(z * w2_ref[0:1, :], axis=1, keepdims=True)
    q = jnp.sum(z * w2_ref[1:2, :], axis=1, keepdims=True)
    out_ref[...] = jnp.concatenate([p, q], axis=1) + b2_ref[...]


_mesh = plsc.VectorSubcoreMesh(core_axis_name="c", subcore_axis_name="s")


@functools.partial(
    pl.kernel,
    out_type=jax.ShapeDtypeStruct((_N_EDGES,), jnp.float32),
    mesh=_mesh,
    scratch_types=[
        pltpu.VMEM((_N_NODES, 2), jnp.float32),
        pltpu.VMEM((_EPW,), jnp.int32),
        pltpu.VMEM((_EPW,), jnp.int32),
        pltpu.VMEM((_EPW,), jnp.float32),
    ],
)
def _edge_sigmoid(pq_hbm, ei_hbm, out_hbm, pq_v, src_v, dst_v, o_v):
    wid = lax.axis_index("s") * _NC + lax.axis_index("c")
    base = wid * _EPW
    pltpu.sync_copy(pq_hbm, pq_v)
    pltpu.sync_copy(ei_hbm.at[0, pl.ds(base, _EPW)], src_v)
    pltpu.sync_copy(ei_hbm.at[1, pl.ds(base, _EPW)], dst_v)
    zero = jnp.zeros((_L,), jnp.int32)
    one = zero + 1

    def body(i, carry):
        off = i * _L
        sv = src_v[pl.ds(off, _L)]
        dv = dst_v[pl.ds(off, _L)]
        pv = plsc.load_gather(pq_v, [sv, zero])
        qv = plsc.load_gather(pq_v, [dv, one])
        x = pv + qv
        o_v[pl.ds(off, _L)] = 1.0 / (1.0 + jnp.exp(-x))
        return carry

    lax.fori_loop(0, _EPW // _L, body, 0)
    pltpu.sync_copy(o_v, out_hbm.at[pl.ds(base, _EPW)])


def kernel(z, edge_index, W, b):
    w2 = jnp.concatenate([W[:, :_D], W[:, _D:]], axis=0)
    b2 = jnp.concatenate([b, jnp.zeros_like(b)]).reshape(1, 2)
    ei = edge_index.astype(jnp.int32)
    pq = pl.pallas_call(
        _pq_body,
        out_shape=jax.ShapeDtypeStruct((_N_NODES, 2), jnp.float32),
    )(z, w2, b2)
    return _edge_sigmoid(pq, ei)


# speedup vs baseline: 26.9674x; 26.9674x over previous
import functools

import jax
import jax.numpy as jnp
from jax import lax
from jax.experimental import pallas as pl
from jax.experimental.pallas import tpu as pltpu
from jax.experimental.pallas import tpu_sc as plsc

_N_NODES = 10000
_N_EDGES = 320000
_D = 128

_NC = 2
_NS = 16
_NW = _NC * _NS
_EPW = _N_EDGES // _NW
_L = 16


def _pq_body(z_ref, w2_ref, b2_ref, out_ref):
    z = z_ref[...]
    p = jnp.sum(z * w2_ref[0:1, :], axis=1, keepdims=True)
    q = jnp.sum(z * w2_ref[1:2, :], axis=1, keepdims=True)
    out_ref[...] = jnp.concatenate([p, q], axis=1) + b2_ref[...]


_mesh = plsc.VectorSubcoreMesh(core_axis_name="c", subcore_axis_name="s")


@functools.partial(
    pl.kernel,
    out_type=jax.ShapeDtypeStruct((_N_EDGES,), jnp.float32),
    mesh=_mesh,
    compiler_params=pltpu.CompilerParams(
        needs_layout_passes=False,
        use_tc_tiling_on_sc=False,
    ),
    scratch_types=[
        pltpu.VMEM((2 * _N_NODES,), jnp.float32),
        pltpu.VMEM((_EPW,), jnp.int32),
        pltpu.VMEM((_EPW,), jnp.int32),
        pltpu.VMEM((_EPW,), jnp.float32),
    ],
)
def _edge_sigmoid(pq_hbm, src_hbm, dst_hbm, out_hbm, pq_v, src_v, dst_v, o_v):
    wid = lax.axis_index("s") * _NC + lax.axis_index("c")
    base = wid * _EPW
    pltpu.sync_copy(pq_hbm, pq_v)
    pltpu.sync_copy(src_hbm.at[pl.ds(base, _EPW)], src_v)
    pltpu.sync_copy(dst_hbm.at[pl.ds(base, _EPW)], dst_v)

    def body(i, carry):
        off = i * _L
        sv = src_v[pl.ds(off, _L)]
        dv = dst_v[pl.ds(off, _L)]
        pv = plsc.load_gather(pq_v, [sv * 2])
        qv = plsc.load_gather(pq_v, [dv * 2 + 1])
        x = pv + qv
        o_v[pl.ds(off, _L)] = 1.0 / (1.0 + jnp.exp(-x))
        return carry

    lax.fori_loop(0, _EPW // _L, body, 0)
    pltpu.sync_copy(o_v, out_hbm.at[pl.ds(base, _EPW)])


def kernel(z, edge_index, W, b):
    w2 = jnp.concatenate([W[:, :_D], W[:, _D:]], axis=0)
    b2 = jnp.concatenate([b, jnp.zeros_like(b)]).reshape(1, 2)
    ei = edge_index.astype(jnp.int32)
    pq = pl.pallas_call(
        _pq_body,
        out_shape=jax.ShapeDtypeStruct((_N_NODES, 2), jnp.float32),
    )(z, w2, b2)
    return _edge_sigmoid(pq.reshape(-1), ei[0], ei[1])


# speedup vs baseline: 33.5241x; 1.2431x over previous
import functools

import jax
import jax.numpy as jnp
from jax import lax
from jax.experimental import pallas as pl
from jax.experimental.pallas import tpu as pltpu
from jax.experimental.pallas import tpu_sc as plsc

_N_NODES = 10000
_N_EDGES = 320000
_D = 128

_NC = 2
_NS = 16
_NW = _NC * _NS
_EPW = _N_EDGES // _NW
_L = 16


def _pq_body(z_ref, w2_ref, b2_ref, out_ref):
    z = z_ref[...]
    p = jnp.sum(z * w2_ref[0:1, :], axis=1, keepdims=True)
    q = jnp.sum(z * w2_ref[1:2, :], axis=1, keepdims=True)
    out_ref[...] = -(jnp.concatenate([p, q], axis=1) + b2_ref[...])


_mesh = plsc.VectorSubcoreMesh(core_axis_name="c", subcore_axis_name="s")


@functools.partial(
    pl.kernel,
    out_type=jax.ShapeDtypeStruct((_N_EDGES,), jnp.float32),
    mesh=_mesh,
    compiler_params=pltpu.CompilerParams(
        needs_layout_passes=False,
        use_tc_tiling_on_sc=False,
    ),
    scratch_types=[
        pltpu.VMEM((2 * _N_NODES,), jnp.float32),
        pltpu.VMEM((_EPW,), jnp.int32),
        pltpu.VMEM((_EPW,), jnp.int32),
        pltpu.VMEM((_EPW,), jnp.float32),
        pltpu.SemaphoreType.DMA,
    ],
)
def _edge_sigmoid(pq_hbm, src_hbm, dst_hbm, out_hbm, pq_v, src_v, dst_v, o_v,
                  sem):
    wid = lax.axis_index("s") * _NC + lax.axis_index("c")
    base = wid * _EPW
    c1 = pltpu.async_copy(pq_hbm, pq_v, sem)
    c2 = pltpu.async_copy(src_hbm.at[pl.ds(base, _EPW)], src_v, sem)
    c3 = pltpu.async_copy(dst_hbm.at[pl.ds(base, _EPW)], dst_v, sem)
    c1.wait()
    c2.wait()
    c3.wait()

    @plsc.parallel_loop(0, _EPW, step=_L, unroll=5)
    def _loop(off):
        sv = src_v[pl.ds(off, _L)]
        dv = dst_v[pl.ds(off, _L)]
        pv = plsc.load_gather(pq_v, [sv * 2])
        qv = plsc.load_gather(pq_v, [dv * 2 + 1])
        o_v[pl.ds(off, _L)] = 1.0 / (1.0 + jnp.exp(pv + qv))

    pltpu.sync_copy(o_v, out_hbm.at[pl.ds(base, _EPW)])


def kernel(z, edge_index, W, b):
    w2 = jnp.concatenate([W[:, :_D], W[:, _D:]], axis=0)
    b2 = jnp.concatenate([b, jnp.zeros_like(b)]).reshape(1, 2)
    ei = edge_index.astype(jnp.int32)
    pq = pl.pallas_call(
        _pq_body,
        out_shape=jax.ShapeDtypeStruct((_N_NODES, 2), jnp.float32),
    )(z, w2, b2)
    return _edge_sigmoid(pq.reshape(-1), ei[0], ei[1])


# speedup vs baseline: 34.9500x; 1.0425x over previous
import functools

import jax
import jax.numpy as jnp
from jax import lax
from jax.experimental import pallas as pl
from jax.experimental.pallas import tpu as pltpu
from jax.experimental.pallas import tpu_sc as plsc

_N_NODES = 10000
_N_EDGES = 320000
_D = 128

_NC = 2
_NS = 16
_NW = _NC * _NS
_EPW = _N_EDGES // _NW
_L = 16


def _pq_body(z_ref, w_ref, b_ref, out_ref):
    z = z_ref[...]
    p = jnp.sum(z * w_ref[0:1, :_D], axis=1, keepdims=True) + b_ref[0]
    q = jnp.sum(z * w_ref[0:1, _D:], axis=1, keepdims=True)
    out_ref[...] = -jnp.concatenate([p, q], axis=1)


_mesh = plsc.VectorSubcoreMesh(core_axis_name="c", subcore_axis_name="s")


@functools.partial(
    pl.kernel,
    out_type=jax.ShapeDtypeStruct((_N_EDGES,), jnp.float32),
    mesh=_mesh,
    compiler_params=pltpu.CompilerParams(
        needs_layout_passes=False,
        use_tc_tiling_on_sc=False,
    ),
    scratch_types=[
        pltpu.VMEM((_N_NODES, 2), jnp.float32),
        pltpu.VMEM((_EPW,), jnp.int32),
        pltpu.VMEM((_EPW,), jnp.int32),
        pltpu.VMEM((_EPW,), jnp.float32),
        pltpu.SemaphoreType.DMA,
    ],
)
def _edge_sigmoid(pq_hbm, ei_hbm, out_hbm, pq_v, src_v, dst_v, o_v, sem):
    wid = lax.axis_index("s") * _NC + lax.axis_index("c")
    base = wid * _EPW
    c1 = pltpu.async_copy(pq_hbm, pq_v, sem)
    c2 = pltpu.async_copy(ei_hbm.at[0, pl.ds(base, _EPW)], src_v, sem)
    c3 = pltpu.async_copy(ei_hbm.at[1, pl.ds(base, _EPW)], dst_v, sem)
    c1.wait()
    c2.wait()
    c3.wait()
    zero = jnp.zeros((_L,), jnp.int32)
    one = zero + 1

    @plsc.parallel_loop(0, _EPW, step=_L, unroll=5)
    def _loop(off):
        sv = src_v[pl.ds(off, _L)]
        dv = dst_v[pl.ds(off, _L)]
        pv = plsc.load_gather(pq_v, [sv, zero])
        qv = plsc.load_gather(pq_v, [dv, one])
        o_v[pl.ds(off, _L)] = 1.0 / (1.0 + jnp.exp(pv + qv))

    pltpu.sync_copy(o_v, out_hbm.at[pl.ds(base, _EPW)])


def kernel(z, edge_index, W, b):
    ei = edge_index.astype(jnp.int32)
    pq = pl.pallas_call(
        _pq_body,
        out_shape=jax.ShapeDtypeStruct((_N_NODES, 2), jnp.float32),
        in_specs=[
            pl.BlockSpec(memory_space=pltpu.VMEM),
            pl.BlockSpec(memory_space=pltpu.VMEM),
            pl.BlockSpec(memory_space=pltpu.SMEM),
        ],
    )(z, W, b)
    return _edge_sigmoid(pq, ei)


# speedup vs baseline: 51.6705x; 1.4784x over previous
import functools

import jax
import jax.numpy as jnp
from jax import lax
from jax.experimental import pallas as pl
from jax.experimental.pallas import tpu as pltpu
from jax.experimental.pallas import tpu_sc as plsc

_N_NODES = 10000
_N_EDGES = 320000
_D = 128

_NC = 2
_NS = 16
_NW = _NC * _NS
_EPW = _N_EDGES // _NW
_L = 16


def _pq_body(z_ref, w_ref, b_ref, p_ref, q_ref):
    w2 = jnp.concatenate([w_ref[0:1, :_D], w_ref[0:1, _D:]], axis=0)
    pq = lax.dot_general(
        w2, z_ref[...], (((1,), (1,)), ((), ())),
        preferred_element_type=jnp.float32,
    )
    p_ref[...] = -(pq[0] + b_ref[0])
    q_ref[...] = -pq[1]


_mesh = plsc.VectorSubcoreMesh(core_axis_name="c", subcore_axis_name="s")


@functools.partial(
    pl.kernel,
    out_type=jax.ShapeDtypeStruct((_N_EDGES,), jnp.float32),
    mesh=_mesh,
    compiler_params=pltpu.CompilerParams(
        needs_layout_passes=False,
        use_tc_tiling_on_sc=False,
    ),
    scratch_types=[
        pltpu.VMEM((_N_NODES,), jnp.float32),
        pltpu.VMEM((_N_NODES,), jnp.float32),
        pltpu.VMEM((_EPW,), jnp.int32),
        pltpu.VMEM((_EPW,), jnp.int32),
        pltpu.VMEM((_EPW,), jnp.float32),
        pltpu.SemaphoreType.DMA,
    ],
)
def _edge_sigmoid(p_hbm, q_hbm, ei_hbm, out_hbm,
                  p_v, q_v, src_v, dst_v, o_v, sem):
    wid = lax.axis_index("s") * _NC + lax.axis_index("c")
    base = wid * _EPW
    c1 = pltpu.async_copy(p_hbm, p_v, sem)
    c2 = pltpu.async_copy(q_hbm, q_v, sem)
    c3 = pltpu.async_copy(ei_hbm.at[0, pl.ds(base, _EPW)], src_v, sem)
    c4 = pltpu.async_copy(ei_hbm.at[1, pl.ds(base, _EPW)], dst_v, sem)
    c1.wait()
    c2.wait()
    c3.wait()
    c4.wait()

    @plsc.parallel_loop(0, _EPW, step=_L, unroll=5)
    def _loop(off):
        sv = src_v[pl.ds(off, _L)]
        dv = dst_v[pl.ds(off, _L)]
        pv = plsc.load_gather(p_v, [sv])
        qv = plsc.load_gather(q_v, [dv])
        o_v[pl.ds(off, _L)] = 1.0 / (1.0 + jnp.exp(pv + qv))

    pltpu.sync_copy(o_v, out_hbm.at[pl.ds(base, _EPW)])


def kernel(z, edge_index, W, b):
    ei = edge_index.astype(jnp.int32)
    p, q = pl.pallas_call(
        _pq_body,
        out_shape=[
            jax.ShapeDtypeStruct((_N_NODES,), jnp.float32),
            jax.ShapeDtypeStruct((_N_NODES,), jnp.float32),
        ],
        in_specs=[
            pl.BlockSpec(memory_space=pltpu.VMEM),
            pl.BlockSpec(memory_space=pltpu.VMEM),
            pl.BlockSpec(memory_space=pltpu.SMEM),
        ],
    )(z, W, b)
    return _edge_sigmoid(p, q, ei)
